# Initial kernel scaffold; baseline (speedup 1.0000x reference)
import jax
import jax.numpy as jnp
from jax.experimental import pallas as pl


def kernel(style_feat, style_skt, content_skt):
    raise NotImplementedError("write your pallas kernel here")


# speedup vs baseline: 1.3188x; 1.3188x over previous
import functools

import numpy as np
import jax
import jax.numpy as jnp
from jax import lax
from jax.experimental import pallas as pl
from jax.experimental.pallas import tpu as pltpu

_HW = 32
_K, _S = 5, 2


def _avg_pool_matrix(n_in, n_out):
    m = np.zeros((n_out, n_in), dtype=np.float32)
    for o in range(n_out):
        s = (o * n_in) // n_out
        e = -(-((o + 1) * n_in) // n_out)
        m[o, s:e] = 1.0 / (e - s)
    return m


def _tail_matrix(hw):
    n1 = (hw - _K) // _S + 1
    n2 = (n1 - _K) // _S + 1
    a = _avg_pool_matrix(n2, hw // 8)
    f = np.zeros((hw, hw), dtype=np.float32)
    for p in range(hw):
        for b in range(n2):
            f[p, _S * _S * b] = a[p % (hw // 8), b]
    return f


def _kron_eye(m, k):
    return np.kron(np.eye(k, dtype=np.float32), m).astype(np.float32)


def _win13(x, axis):
    n = x.shape[axis]
    m = jnp.maximum(x, pltpu.roll(x, shift=n - 1, axis=axis))
    m = jnp.maximum(m, pltpu.roll(m, shift=n - 2, axis=axis))
    m = jnp.maximum(m, pltpu.roll(m, shift=n - 4, axis=axis))
    return jnp.maximum(m, pltpu.roll(m, shift=n - 5, axis=axis))


def _masks_kernel(skt_ref, pt_ref, bpt_ref, o_ref, *, B, hw):
    f32 = jnp.float32
    y = jnp.dot(skt_ref[0], pt_ref[...], preferred_element_type=f32)
    z = lax.dot_general(y, bpt_ref[...], (((0,), (0,)), ((), ())),
                        preferred_element_type=f32)
    lo = jnp.min(z)
    hi = jnp.max(z)
    r = (z - lo) / jnp.maximum(hi - lo, 1e-12)
    for b in range(B):
        o_ref[0, b * hw:(b + 1) * hw, :] = r[:, b * hw:(b + 1) * hw]


def _transfer_kernel(sf_ref, msk_ref, pt_ref, bpt_ref, f_ref, gt_ref, o_ref,
                     *, SPS, Cb, H, hw):
    f32 = jnp.float32
    m_t = jnp.concatenate([msk_ref[0]] * Cb, axis=1)
    cm_t = jnp.concatenate([msk_ref[1]] * Cb, axis=1)

    x = sf_ref[0]
    y = jnp.dot(x, pt_ref[...], preferred_element_type=f32)

    f_m = f_ref[...]
    gt_m = gt_ref[...]
    bpt = bpt_ref[...]
    for s in range(SPS):
        ys = y[s * Cb * H:(s + 1) * Cb * H]
        g = lax.dot_general(ys, bpt, (((0,), (0,)), ((), ())),
                            preferred_element_type=f32)
        e = g * m_t
        w = jnp.concatenate([e, g - e], axis=1)
        w = _win13(_win13(w, 1), 0)
        u = jnp.dot(f_m, w, preferred_element_type=f32)
        v = jnp.dot(u, gt_m, preferred_element_type=f32)
        eo = v[:, :Cb * hw]
        po = v[:, Cb * hw:]
        o_ref[0, s] = po + (eo - po) * cm_t


@jax.jit
def _featmap_transfer(style_feat, style_skt, content_skt):
    B, C, H, W = style_feat.shape
    hw = _HW

    Cb = max(1, 128 // hw)
    while C % Cb:
        Cb //= 2
    nG = C // Cb

    max_sps = max(1, min(8, (2 * 1024 * 1024) // (Cb * H * W * 4)))
    if B == 1:
        max_sps = min(max_sps, max(1, nG // 2))
    SPS = 1
    for d in range(1, nG + 1):
        if nG % d == 0 and d <= max_sps:
            SPS = d
    nGsteps = nG // SPS

    p = _avg_pool_matrix(H, hw)
    f = _tail_matrix(hw)
    pt_f32 = jnp.asarray(p.T)
    bbt = jnp.asarray(_kron_eye(p.T, B))
    bpt_f32 = jnp.asarray(_kron_eye(p.T, Cb))
    f_f32 = jnp.asarray(f)
    gt_f32 = jnp.asarray(_kron_eye(f.T, 2 * Cb))

    skts = jnp.stack([style_skt, content_skt], axis=0).reshape(2, B * H, W)
    masks = pl.pallas_call(
        functools.partial(_masks_kernel, B=B, hw=hw),
        out_shape=jax.ShapeDtypeStruct((2, B * hw, hw), jnp.float32),
        grid=(2,),
        in_specs=[pl.BlockSpec((1, B * H, W), lambda s: (s, 0, 0)),
                  pl.BlockSpec((H, hw), lambda s: (0, 0)),
                  pl.BlockSpec((B * H, B * hw), lambda s: (0, 0))],
        out_specs=pl.BlockSpec((1, B * hw, hw), lambda s: (s, 0, 0)),
        compiler_params=pltpu.CompilerParams(
            dimension_semantics=("parallel",)),
    )(skts, pt_f32, bbt)

    sf2 = style_feat.reshape(B, C * H, W)
    out_packed = pl.pallas_call(
        functools.partial(_transfer_kernel, SPS=SPS, Cb=Cb, H=H, hw=hw),
        out_shape=jax.ShapeDtypeStruct((B, nG, hw, Cb * hw), jnp.float32),
        grid=(B, nGsteps),
        in_specs=[
            pl.BlockSpec((1, SPS * Cb * H, W), lambda b, g: (b, g, 0)),
            pl.BlockSpec((2, hw, hw), lambda b, g: (0, b, 0)),
            pl.BlockSpec((H, hw), lambda b, g: (0, 0)),
            pl.BlockSpec((Cb * H, Cb * hw), lambda b, g: (0, 0)),
            pl.BlockSpec((hw, hw), lambda b, g: (0, 0)),
            pl.BlockSpec((2 * Cb * hw, 2 * Cb * hw), lambda b, g: (0, 0)),
        ],
        out_specs=pl.BlockSpec((1, SPS, hw, Cb * hw), lambda b, g: (b, g, 0, 0)),
        compiler_params=pltpu.CompilerParams(
            dimension_semantics=("parallel", "parallel"),
            vmem_limit_bytes=32 * 1024 * 1024),
    )(sf2, masks, pt_f32, bpt_f32, f_f32, gt_f32)

    out = out_packed.reshape(B, nG, hw, Cb, hw).transpose(0, 1, 3, 4, 2)
    return out.reshape(B, C, hw, hw)


def kernel(style_feat, style_skt, content_skt):
    return _featmap_transfer(style_feat, style_skt, content_skt)
